# baseline (speedup 1.0000x reference)
import functools

import numpy as np
import jax
import jax.numpy as jnp
from jax.experimental import pallas as pl
from jax.experimental.pallas import tpu as pltpu
from jax.experimental.pallas import tpu_sc as plsc

N_LEVELS = 10
N_FEATS = 8
T = 2 ** 20
BASE_RES = 16
MAX_RES = 1024
FEAT_DIM = 128
_SCALE = np.exp((np.log(MAX_RES) - np.log(BASE_RES)) / (N_LEVELS - 1))
RES = [int(np.floor(BASE_RES * _SCALE ** l)) for l in range(N_LEVELS)]
TSIZE = [min(T, (r + 1) ** 2) for r in RES]
PRIME_I32 = np.uint32(2654435761).astype(np.int32)
HASH_MASK = T - 1

BR = 2048
NBL = [(ts + BR - 1) // BR for ts in TSIZE[:9]]
SBLK = np.concatenate([[0], np.cumsum(NBL)]).astype(np.int32)
POFF = (SBLK[:9] * BR).astype(np.int32)
MEGA_ROWS = int(SBLK[9]) * BR

N_PSLOT = 18
N_9SLOT = 4
N_W = 40
GCOLS = N_PSLOT * 16 + N_9SLOT * 8

BP = 512
BM = 1024
GW = 2048


def _build_maps():
    E = np.zeros((N_W, GCOLS), np.float32)
    rowmap = np.zeros((GCOLS,), np.int32)
    for s in range(N_PSLOT):
        l = s if s < 9 else s - 9
        yc = 0 if s < 9 else 1
        for xc in range(2):
            k = l + 9 * (2 * yc + xc)
            E[k, s * 16 + 8 * xc: s * 16 + 8 * xc + 8] = 1.0
        for j in range(16):
            rowmap[s * 16 + j] = l * 8 + (j % 8)
    for c4 in range(N_9SLOT):
        base = N_PSLOT * 16 + c4 * 8
        E[36 + c4, base: base + 8] = 1.0
        for j in range(8):
            rowmap[base + j] = 72 + j
    return E, rowmap


E_NP, ROWMAP_NP = _build_maps()


def _prep_body(xs_ref, ys_ref, resf_ref, ic_ref, idxp_ref, idx9_ref, w_ref):
    x = xs_ref[0]
    y = ys_ref[0]
    resv = resf_ref[...]
    rp1v = ic_ref[:, 0:1]
    poffv = ic_ref[:, 1:2]

    px = x * resv
    py = y * resv
    x0f = jnp.clip(jnp.floor(px), 0.0, resv - 1.0)
    y0f = jnp.clip(jnp.floor(py), 0.0, resv - 1.0)
    fx = px - x0f
    fy = py - y0f
    xi = x0f.astype(jnp.int32)
    yi = y0f.astype(jnp.int32)

    base0 = xi + yi * rp1v + poffv
    base1 = base0 + rp1v
    xi9 = xi[9:10]
    yh0 = yi[9:10] * int(PRIME_I32)
    yh1 = (yi[9:10] + 1) * int(PRIME_I32)
    h00 = (xi9 ^ yh0) & HASH_MASK
    h10 = ((xi9 + 1) ^ yh0) & HASH_MASK
    h01 = (xi9 ^ yh1) & HASH_MASK
    h11 = ((xi9 + 1) ^ yh1) & HASH_MASK
    idx22 = jnp.transpose(jnp.concatenate(
        [base0[0:9], base1[0:9], h00, h10, h01, h11], axis=0))
    idxp_ref[...] = idx22[:, :N_PSLOT]
    idx9_ref[...] = idx22[:, N_PSLOT:]

    wx0 = 1.0 - fx
    wy0 = 1.0 - fy
    a00 = wx0 * wy0
    a10 = fx * wy0
    a01 = wx0 * fy
    a11 = fx * fy
    w40 = jnp.concatenate(
        [a00[0:9], a10[0:9], a01[0:9], a11[0:9],
         a00[9:10], a10[9:10], a01[9:10], a11[9:10]], axis=0)
    w_ref[...] = jnp.transpose(w40)


def _prep_call(xs3, ys3):
    nb = xs3.shape[0]
    resf = jnp.asarray(np.array(RES, np.float32).reshape(N_LEVELS, 1))
    ic = jnp.asarray(np.stack(
        [np.array([r + 1 for r in RES], np.int32),
         np.concatenate([POFF, [0]]).astype(np.int32)], axis=1))
    return pl.pallas_call(
        _prep_body,
        grid=(nb,),
        in_specs=[pl.BlockSpec((1, 1, BP), lambda i: (i, 0, 0)),
                  pl.BlockSpec((1, 1, BP), lambda i: (i, 0, 0)),
                  pl.BlockSpec((N_LEVELS, 1), lambda i: (0, 0)),
                  pl.BlockSpec((N_LEVELS, 2), lambda i: (0, 0))],
        out_specs=[pl.BlockSpec((BP, N_PSLOT), lambda i: (i, 0)),
                   pl.BlockSpec((BP, N_9SLOT), lambda i: (i, 0)),
                   pl.BlockSpec((BP, N_W), lambda i: (i, 0))],
        out_shape=[jax.ShapeDtypeStruct((nb * BP, N_PSLOT), jnp.int32),
                   jax.ShapeDtypeStruct((nb * BP, N_9SLOT), jnp.int32),
                   jax.ShapeDtypeStruct((nb * BP, N_W), jnp.float32)],
    )(xs3, ys3, resf, ic)


def _pairs_body(*refs):
    o_ref = refs[-1]
    i = pl.program_id(0)
    for l in range(9):
        a_ref, b_ref = refs[2 * l], refs[2 * l + 1]

        @pl.when(jnp.logical_and(i >= int(SBLK[l]), i < int(SBLK[l + 1])))
        def _(a_ref=a_ref, b_ref=b_ref):
            a = jnp.transpose(a_ref[...])
            b = jnp.transpose(b_ref[...])
            shifted = jnp.concatenate([a[1:], b[0:1]], axis=0)
            o_ref[...] = jnp.concatenate([a, shifted], axis=1)


def _build_pairs(tTs):
    in_specs = []
    for l in range(9):
        nbl, sb = NBL[l], int(SBLK[l])
        in_specs.append(pl.BlockSpec(
            (8, BR), lambda i, sb=sb, nbl=nbl: (0, jnp.clip(i - sb, 0, nbl - 1))))
        in_specs.append(pl.BlockSpec(
            (8, BR), lambda i, sb=sb, nbl=nbl: (0, jnp.clip(i - sb + 1, 0, nbl - 1))))
    args = []
    for t in tTs:
        args += [t, t]
    return pl.pallas_call(
        _pairs_body,
        grid=(int(SBLK[9]),),
        in_specs=in_specs,
        out_specs=pl.BlockSpec((BR, 16), lambda i: (i, 0)),
        out_shape=jax.ShapeDtypeStruct((MEGA_ROWS, 16), jnp.float32),
    )(*args)


def _rowmajor_body(aT_ref, o_ref):
    o_ref[...] = jnp.transpose(aT_ref[...])


def _to_rowmajor(tT):
    rows = tT.shape[1]
    nb = rows // 8192
    return pl.pallas_call(
        _rowmajor_body,
        grid=(nb,),
        in_specs=[pl.BlockSpec((8, 8192), lambda i: (0, i))],
        out_specs=pl.BlockSpec((8192, 8), lambda i: (i, 0)),
        out_shape=jax.ShapeDtypeStruct((rows, 8), jnp.float32),
    )(tT)


def _gather_sc(megap, t9, idxp2, idx92):
    gp_rows = idxp2.shape[0] * idxp2.shape[1]
    g9_rows = idx92.shape[0] * idx92.shape[1]
    gridp, grid9 = idxp2.shape[0], idx92.shape[0]
    wp, w9 = idxp2.shape[1], idx92.shape[1]
    mesh = plsc.VectorSubcoreMesh(core_axis_name="c", subcore_axis_name="s")

    @functools.partial(
        pl.kernel,
        out_type=[jax.ShapeDtypeStruct((gp_rows, 16), jnp.float32),
                  jax.ShapeDtypeStruct((g9_rows, 8), jnp.float32)],
        mesh=mesh,
        compiler_params=pltpu.CompilerParams(use_tc_tiling_on_sc=False))
    def gather_kernel(megap_hbm, t9_hbm, idxp_hbm, idx9_hbm, gp_hbm, g9_hbm):
        def bodyp(i_vmem, o_vmem):
            pltpu.sync_copy(megap_hbm.at[i_vmem.at[0]], o_vmem)

        pltpu.emit_pipeline(
            bodyp,
            grid=(gridp,),
            in_specs=[pl.BlockSpec((1, wp), lambda i: (i, 0))],
            out_specs=[pl.BlockSpec((wp, 16), lambda i: (i, 0))],
            core_axis_name=("c", "s"),
            dimension_semantics=(pltpu.PARALLEL,),
        )(idxp_hbm, gp_hbm)

        def body9(i_vmem, o_vmem):
            pltpu.sync_copy(t9_hbm.at[i_vmem.at[0]], o_vmem)

        pltpu.emit_pipeline(
            body9,
            grid=(grid9,),
            in_specs=[pl.BlockSpec((1, w9), lambda i: (i, 0))],
            out_specs=[pl.BlockSpec((w9, 8), lambda i: (i, 0))],
            core_axis_name=("c", "s"),
            dimension_semantics=(pltpu.PARALLEL,),
        )(idx9_hbm, g9_hbm)

    return gather_kernel(megap, t9, idxp2, idx92)


def _mlp_body(gp_ref, g9_ref, w_ref, e_ref, w1r_ref, b1_ref, w2_ref, b2_ref,
              o_ref):
    wexp = jnp.dot(w_ref[...], e_ref[...], preferred_element_type=jnp.float32)
    t = jnp.concatenate([gp_ref[...], g9_ref[...]], axis=1) * wexp
    h = jnp.dot(t, w1r_ref[...], preferred_element_type=jnp.float32) + b1_ref[...]
    h = jnp.maximum(h, 0.0)
    o_ref[...] = jnp.dot(h, w2_ref[...], preferred_element_type=jnp.float32) + b2_ref[...]


def _mlp_call(gpm, g9m, w, e, w1r, b1, w2, b2):
    n = gpm.shape[0]
    nb = n // BM
    hid = w1r.shape[1]
    return pl.pallas_call(
        _mlp_body,
        grid=(nb,),
        in_specs=[pl.BlockSpec((BM, N_PSLOT * 16), lambda i: (i, 0)),
                  pl.BlockSpec((BM, N_9SLOT * 8), lambda i: (i, 0)),
                  pl.BlockSpec((BM, N_W), lambda i: (i, 0)),
                  pl.BlockSpec((N_W, GCOLS), lambda i: (0, 0)),
                  pl.BlockSpec((GCOLS, hid), lambda i: (0, 0)),
                  pl.BlockSpec((1, hid), lambda i: (0, 0)),
                  pl.BlockSpec((hid, FEAT_DIM), lambda i: (0, 0)),
                  pl.BlockSpec((1, FEAT_DIM), lambda i: (0, 0))],
        out_specs=pl.BlockSpec((BM, FEAT_DIM), lambda i: (i, 0)),
        out_shape=jax.ShapeDtypeStruct((n, FEAT_DIM), jnp.float32),
    )(gpm, g9m, w, e, w1r, b1, w2, b2)


def kernel(coords, table0, table1, table2, table3, table4, table5, table6,
           table7, table8, table9, W1, b1, W2, b2):
    tables = [table0, table1, table2, table3, table4, table5, table6,
              table7, table8, table9]
    n = coords.shape[0]
    nb = n // BP

    megap = _build_pairs([jnp.transpose(t) for t in tables[:9]])
    t9row = _to_rowmajor(jnp.transpose(tables[9]))

    xs3 = coords[:, 0].reshape(nb, 1, BP)
    ys3 = coords[:, 1].reshape(nb, 1, BP)
    idxp, idx9, w = _prep_call(xs3, ys3)

    e = jnp.asarray(E_NP)
    w1r = jnp.take(W1, jnp.asarray(ROWMAP_NP), axis=0)
    b1r = b1.reshape(1, -1)
    b2r = b2.reshape(1, -1)

    nh = n // 2
    outs = []
    for c in range(2):
        sl = slice(c * nh, (c + 1) * nh)
        idxp2 = idxp[sl].reshape(nh * N_PSLOT // GW, GW)
        idx92 = idx9[sl].reshape(nh * N_9SLOT // GW, GW)
        gp, g9 = _gather_sc(megap, t9row, idxp2, idx92)
        gpm = gp.reshape(nh, N_PSLOT * 16)
        g9m = g9.reshape(nh, N_9SLOT * 8)
        outs.append(_mlp_call(gpm, g9m, w[sl], e, w1r, b1r, W2, b2r))
    return jnp.concatenate(outs, axis=0)


# speedup vs baseline: 1.2233x; 1.2233x over previous
import functools

import numpy as np
import jax
import jax.numpy as jnp
from jax.experimental import pallas as pl
from jax.experimental.pallas import tpu as pltpu
from jax.experimental.pallas import tpu_sc as plsc

N_LEVELS = 10
N_FEATS = 8
T = 2 ** 20
BASE_RES = 16
MAX_RES = 1024
FEAT_DIM = 128
_SCALE = np.exp((np.log(MAX_RES) - np.log(BASE_RES)) / (N_LEVELS - 1))
RES = [int(np.floor(BASE_RES * _SCALE ** l)) for l in range(N_LEVELS)]
TSIZE = [min(T, (r + 1) ** 2) for r in RES]
PRIME_I32 = np.uint32(2654435761).astype(np.int32)
HASH_MASK = T - 1

POFF = np.concatenate([[0], np.cumsum(TSIZE[:9])]).astype(np.int32)
OFF9 = int(POFF[9])
MEGA_ROWS = OFF9 + TSIZE[9]

N_SLOTS = 22
N_W = 40
GCOLS = N_SLOTS * 16

BP = 512
BM = 1024
GW = 2048


def _build_maps():
    E = np.zeros((N_W, GCOLS), np.float32)
    rowmap = np.zeros((GCOLS,), np.int32)
    rowmask = np.zeros((GCOLS, 1), np.float32)
    for s in range(18):
        l = s if s < 9 else s - 9
        yc = 0 if s < 9 else 1
        for xc in range(2):
            k = l + 9 * (2 * yc + xc)
            E[k, s * 16 + 8 * xc: s * 16 + 8 * xc + 8] = 1.0
        for j in range(16):
            rowmap[s * 16 + j] = l * 8 + (j % 8)
            rowmask[s * 16 + j] = 1.0
    for c4 in range(4):
        s = 18 + c4
        E[36 + c4, s * 16: s * 16 + 8] = 1.0
        for j in range(8):
            rowmap[s * 16 + j] = 72 + j
            rowmask[s * 16 + j] = 1.0
    return E, rowmap, rowmask


E_NP, ROWMAP_NP, ROWMASK_NP = _build_maps()


def _prep_body(xs_ref, ys_ref, resf_ref, ic_ref, idx_ref, w_ref):
    x = xs_ref[0]
    y = ys_ref[0]
    resv = resf_ref[...]
    rp1v = ic_ref[:, 0:1]
    poffv = ic_ref[:, 1:2]

    px = x * resv
    py = y * resv
    x0f = jnp.clip(jnp.floor(px), 0.0, resv - 1.0)
    y0f = jnp.clip(jnp.floor(py), 0.0, resv - 1.0)
    fx = px - x0f
    fy = py - y0f
    xi = x0f.astype(jnp.int32)
    yi = y0f.astype(jnp.int32)

    base0 = xi + yi * rp1v + poffv
    base1 = base0 + rp1v
    xi9 = xi[9:10]
    yh0 = yi[9:10] * int(PRIME_I32)
    yh1 = (yi[9:10] + 1) * int(PRIME_I32)
    h00 = ((xi9 ^ yh0) & HASH_MASK) + OFF9
    h10 = (((xi9 + 1) ^ yh0) & HASH_MASK) + OFF9
    h01 = ((xi9 ^ yh1) & HASH_MASK) + OFF9
    h11 = (((xi9 + 1) ^ yh1) & HASH_MASK) + OFF9
    idx22 = jnp.concatenate(
        [base0[0:9], base1[0:9], h00, h10, h01, h11], axis=0)
    idx_ref[...] = jnp.transpose(idx22)

    wx0 = 1.0 - fx
    wy0 = 1.0 - fy
    a00 = wx0 * wy0
    a10 = fx * wy0
    a01 = wx0 * fy
    a11 = fx * fy
    w40 = jnp.concatenate(
        [a00[0:9], a10[0:9], a01[0:9], a11[0:9],
         a00[9:10], a10[9:10], a01[9:10], a11[9:10]], axis=0)
    w_ref[...] = jnp.transpose(w40)


def _prep_call(xs3, ys3):
    nb = xs3.shape[0]
    resf = jnp.asarray(np.array(RES, np.float32).reshape(N_LEVELS, 1))
    ic = jnp.asarray(np.stack(
        [np.array([r + 1 for r in RES], np.int32),
         np.concatenate([POFF[:9], [0]]).astype(np.int32)], axis=1))
    return pl.pallas_call(
        _prep_body,
        grid=(nb,),
        in_specs=[pl.BlockSpec((1, 1, BP), lambda i: (i, 0, 0)),
                  pl.BlockSpec((1, 1, BP), lambda i: (i, 0, 0)),
                  pl.BlockSpec((N_LEVELS, 1), lambda i: (0, 0)),
                  pl.BlockSpec((N_LEVELS, 2), lambda i: (0, 0))],
        out_specs=[pl.BlockSpec((BP, N_SLOTS), lambda i: (i, 0)),
                   pl.BlockSpec((BP, N_W), lambda i: (i, 0))],
        out_shape=[jax.ShapeDtypeStruct((nb * BP, N_SLOTS), jnp.int32),
                   jax.ShapeDtypeStruct((nb * BP, N_W), jnp.float32)],
    )(xs3, ys3, resf, ic)


def _gather_sc(mega, idx2):
    g_rows, w = idx2.shape[0] * idx2.shape[1], idx2.shape[1]
    grid = idx2.shape[0]
    mesh = plsc.VectorSubcoreMesh(core_axis_name="c", subcore_axis_name="s")

    @functools.partial(
        pl.kernel,
        out_type=jax.ShapeDtypeStruct((g_rows, 16), jnp.float32),
        mesh=mesh,
        compiler_params=pltpu.CompilerParams(use_tc_tiling_on_sc=False))
    def gather_kernel(mega_hbm, idx_hbm, g_hbm):
        def body(i_vmem, o_vmem):
            pltpu.sync_copy(mega_hbm.at[i_vmem.at[0]], o_vmem)

        pltpu.emit_pipeline(
            body,
            grid=(grid,),
            in_specs=[pl.BlockSpec((1, w), lambda i: (i, 0))],
            out_specs=[pl.BlockSpec((w, 16), lambda i: (i, 0))],
            core_axis_name=("c", "s"),
            dimension_semantics=(pltpu.PARALLEL,),
        )(idx_hbm, g_hbm)

    return gather_kernel(mega, idx2)


def _mlp_body(g_ref, w_ref, e_ref, w1r_ref, b1_ref, w2_ref, b2_ref, o_ref):
    wexp = jnp.dot(w_ref[...], e_ref[...], preferred_element_type=jnp.float32)
    t = g_ref[...] * wexp
    h = jnp.dot(t, w1r_ref[...], preferred_element_type=jnp.float32) + b1_ref[...]
    h = jnp.maximum(h, 0.0)
    o_ref[...] = jnp.dot(h, w2_ref[...], preferred_element_type=jnp.float32) + b2_ref[...]


def _mlp_call(gm, w, e, w1r, b1, w2, b2):
    n = gm.shape[0]
    nb = n // BM
    hid = w1r.shape[1]
    return pl.pallas_call(
        _mlp_body,
        grid=(nb,),
        in_specs=[pl.BlockSpec((BM, GCOLS), lambda i: (i, 0)),
                  pl.BlockSpec((BM, N_W), lambda i: (i, 0)),
                  pl.BlockSpec((N_W, GCOLS), lambda i: (0, 0)),
                  pl.BlockSpec((GCOLS, hid), lambda i: (0, 0)),
                  pl.BlockSpec((1, hid), lambda i: (0, 0)),
                  pl.BlockSpec((hid, FEAT_DIM), lambda i: (0, 0)),
                  pl.BlockSpec((1, FEAT_DIM), lambda i: (0, 0))],
        out_specs=pl.BlockSpec((BM, FEAT_DIM), lambda i: (i, 0)),
        out_shape=jax.ShapeDtypeStruct((n, FEAT_DIM), jnp.float32),
    )(gm, w, e, w1r, b1, w2, b2)


def kernel(coords, table0, table1, table2, table3, table4, table5, table6,
           table7, table8, table9, W1, b1, W2, b2):
    tables = [table0, table1, table2, table3, table4, table5, table6,
              table7, table8, table9]
    n = coords.shape[0]
    nb = n // BP

    parts = [jnp.concatenate([t, jnp.roll(t, -1, axis=0)], axis=1)
             for t in tables[:9]]
    parts.append(jnp.concatenate([tables[9], jnp.zeros_like(tables[9])], axis=1))
    mega = jnp.concatenate(parts, axis=0)

    xs3 = coords[:, 0].reshape(nb, 1, BP)
    ys3 = coords[:, 1].reshape(nb, 1, BP)
    idx, w = _prep_call(xs3, ys3)

    idx2 = idx.reshape(n * N_SLOTS // GW, GW)

    g = _gather_sc(mega, idx2)
    gm = g.reshape(n, GCOLS)

    e = jnp.asarray(E_NP)
    w1r = jnp.take(W1, jnp.asarray(ROWMAP_NP), axis=0) * jnp.asarray(ROWMASK_NP)
    out = _mlp_call(gm, w, e, w1r, b1.reshape(1, -1), W2, b2.reshape(1, -1))
    return out
